# baseline (speedup 1.0000x reference)
import jax
import jax.numpy as jnp
from jax import lax
from jax.experimental import pallas as pl
from jax.experimental.pallas import tpu as pltpu
from jax.experimental.pallas import tpu_sc as plsc

NC = 2
NS = 16
NW = NC * NS
CHUNK = 80
NBUF = 4
NIDX = 8
F_SLOW = 0.49
DEG_W = 16


def _pad_rows(n):
    return ((n + NS * 8 - 1) // (NS * 8)) * (NS * 8)


def _splits(e):
    chunks = e // CHUNK
    assert chunks * CHUNK == e
    per_pair = chunks // NS
    assert per_pair * NS == chunks
    n1 = max(int(round(per_pair * F_SLOW)), 2)
    n0 = per_pair - n1
    return n0, n1


def _make_spmm(n_pad, e, d):
    rows_per_tile = n_pad // NS
    n0, n1 = _splits(e)

    mesh = plsc.VectorSubcoreMesh(core_axis_name="c", subcore_axis_name="s")
    out_type = [jax.ShapeDtypeStruct((NC, n_pad, d), jnp.float32)]
    scratch = [
        pltpu.VMEM_SHARED((n_pad, d), jnp.float32),
        pltpu.VMEM((NIDX, CHUNK), jnp.int32),
        pltpu.VMEM((NIDX, CHUNK), jnp.int32),
        pltpu.VMEM((NBUF, CHUNK, d), jnp.float32),
        pltpu.SemaphoreType.DMA((NIDX,)),
        pltpu.SemaphoreType.DMA((NIDX,)),
        pltpu.SemaphoreType.DMA((NBUF,)),
        pltpu.SemaphoreType.DMA((NBUF,)),
    ]

    def body(x_hbm, src_hbm, dst_hbm, z_hbm, out_hbm,
             acc, sidx_v, didx_v, rows_v, xs, ys, gs, cs):
        c = lax.axis_index("c")
        s = lax.axis_index("s")
        row0 = s * rows_per_tile
        nb = jnp.where(c == 0, n0, n1)
        chunk0 = jnp.where(c == 0, s * n0, NS * n0 + s * n1)
        pltpu.sync_copy(z_hbm, acc.at[pl.ds(row0, rows_per_tile)])
        plsc.subcore_barrier()

        def idx_load(chunk, slot):
            base = (chunk0 + jnp.minimum(chunk, nb - 1)) * CHUNK
            pltpu.make_async_copy(
                src_hbm.at[pl.ds(base, CHUNK)], sidx_v.at[slot],
                xs.at[slot]).start()
            pltpu.make_async_copy(
                dst_hbm.at[pl.ds(base, CHUNK)], didx_v.at[slot],
                ys.at[slot]).start()

        def idx_wait(slot):
            pltpu.make_async_copy(
                src_hbm.at[pl.ds(0, CHUNK)], sidx_v.at[slot],
                xs.at[slot]).wait()
            pltpu.make_async_copy(
                dst_hbm.at[pl.ds(0, CHUNK)], didx_v.at[slot],
                ys.at[slot]).wait()

        def gather_start(slot8, buf):
            pltpu.make_async_copy(
                x_hbm.at[sidx_v.at[slot8]], rows_v.at[buf],
                gs.at[buf]).start()

        def gather_wait(buf):
            pltpu.make_async_copy(
                x_hbm.at[sidx_v.at[0]], rows_v.at[buf], gs.at[buf]).wait()

        def scatter_start(slot8, buf):
            pltpu.make_async_copy(
                rows_v.at[buf], acc.at[didx_v.at[slot8]],
                cs.at[buf]).start(add=True)

        def scatter_wait(buf):
            pltpu.make_async_copy(
                rows_v.at[buf], acc.at[didx_v.at[0]], cs.at[buf]).wait()

        for k in range(6):
            idx_load(k, k)
        idx_wait(0)
        gather_start(0, 0)
        idx_wait(1)
        gather_start(1, 1)
        for i in (0, 1):
            gather_wait(i)
            scatter_start(i, i)
            idx_load(i + 6, i + 6)
            idx_wait(i + 2)
            gather_start(i + 2, i + 2)

        def step(i, carry):
            b = lax.rem(i, NBUF)
            gather_wait(b)
            scatter_start(lax.rem(i, NIDX), b)
            scatter_wait(lax.rem(i + 2, NBUF))
            idx_load(i + 6, lax.rem(i + 6, NIDX))
            idx_wait(lax.rem(i + 2, NIDX))
            gather_start(lax.rem(i + 2, NIDX), lax.rem(i + 2, NBUF))
            return carry

        lax.fori_loop(2, nb, step, 0)
        scatter_wait(lax.rem(nb - 2, NBUF))
        scatter_wait(lax.rem(nb - 1, NBUF))
        gather_wait(lax.rem(nb, NBUF))
        gather_wait(lax.rem(nb + 1, NBUF))
        for k in range(2, 6):
            idx_wait(lax.rem(nb + k, NIDX))
        plsc.subcore_barrier()
        pltpu.sync_copy(acc.at[pl.ds(row0, rows_per_tile)],
                        out_hbm.at[c, pl.ds(row0, rows_per_tile)])

    return pl.kernel(body, out_type=out_type, mesh=mesh, scratch_types=scratch,
                     compiler_params=pltpu.CompilerParams(use_tc_tiling_on_sc=False))


def _make_deg(n_pad, e, d):
    rows_per_tile = n_pad // NS
    nb = e // (CHUNK * NW)
    assert nb * CHUNK * NW == e and nb >= 2

    mesh = plsc.VectorSubcoreMesh(core_axis_name="c", subcore_axis_name="s")
    out_type = [jax.ShapeDtypeStruct((NC, n_pad, DEG_W), jnp.float32)]
    scratch = [
        pltpu.VMEM_SHARED((n_pad, DEG_W), jnp.float32),
        pltpu.VMEM((NIDX, CHUNK), jnp.int32),
        pltpu.VMEM((CHUNK, DEG_W), jnp.float32),
        pltpu.SemaphoreType.DMA((NIDX,)),
        pltpu.SemaphoreType.DMA((NBUF,)),
    ]

    def body(dst_hbm, z_hbm, ones_hbm, out_hbm, dacc, idx_v, ones_v, xs, cs):
        c = lax.axis_index("c")
        s = lax.axis_index("s")
        wid = s * NC + c
        row0 = s * rows_per_tile
        chunk0 = wid * nb
        pltpu.sync_copy(z_hbm, dacc.at[pl.ds(row0, rows_per_tile)])
        pltpu.sync_copy(ones_hbm, ones_v)
        plsc.subcore_barrier()

        def idx_load(chunk, slot):
            base = (chunk0 + jnp.minimum(chunk, nb - 1)) * CHUNK
            pltpu.make_async_copy(
                dst_hbm.at[pl.ds(base, CHUNK)], idx_v.at[slot],
                xs.at[slot]).start()

        def idx_wait(slot):
            pltpu.make_async_copy(
                dst_hbm.at[pl.ds(0, CHUNK)], idx_v.at[slot],
                xs.at[slot]).wait()

        def scatter_start(slot8, buf):
            pltpu.make_async_copy(
                ones_v, dacc.at[idx_v.at[slot8]],
                cs.at[buf]).start(add=True)

        def scatter_wait(buf):
            pltpu.make_async_copy(
                ones_v, dacc.at[idx_v.at[0]], cs.at[buf]).wait()

        for k in range(6):
            idx_load(k, k)
        for i in (0, 1):
            idx_wait(i)
            scatter_start(i, i)
            idx_load(i + 6, i + 6)

        def step(i, carry):
            idx_wait(lax.rem(i, NIDX))
            scatter_start(lax.rem(i, NIDX), lax.rem(i, NBUF))
            scatter_wait(lax.rem(i + 2, NBUF))
            idx_load(i + 6, lax.rem(i + 6, NIDX))
            return carry

        lax.fori_loop(2, nb, step, 0)
        scatter_wait(lax.rem(nb - 2, NBUF))
        scatter_wait(lax.rem(nb - 1, NBUF))
        for k in range(0, 6):
            idx_wait(lax.rem(nb + k, NIDX))
        plsc.subcore_barrier()
        pltpu.sync_copy(dacc.at[pl.ds(row0, rows_per_tile)],
                        out_hbm.at[c, pl.ds(row0, rows_per_tile)])

    return pl.kernel(body, out_type=out_type, mesh=mesh, scratch_types=scratch,
                     compiler_params=pltpu.CompilerParams(use_tc_tiling_on_sc=False))


def _tc_layer1(p, dacc, w, n_out, bn=1000):
    d = p.shape[2]
    h = w.shape[1]
    n = n_out

    def body(p_ref, d_ref, w_ref, o_ref):
        agg = p_ref[0] + p_ref[1]
        deg = d_ref[0, :, 0] + d_ref[1, :, 0]
        inv = 1.0 / jnp.maximum(deg, 1.0)
        aggn = agg * inv[:, None]
        o_ref[...] = jnp.maximum(
            jnp.dot(aggn, w_ref[...], preferred_element_type=jnp.float32), 0.0)

    return pl.pallas_call(
        body,
        grid=(n // bn,),
        in_specs=[
            pl.BlockSpec((NC, bn, d), lambda i: (0, i, 0)),
            pl.BlockSpec((NC, bn, DEG_W), lambda i: (0, i, 0)),
            pl.BlockSpec((d, h), lambda i: (0, 0)),
        ],
        out_specs=pl.BlockSpec((bn, h), lambda i: (i, 0)),
        out_shape=jax.ShapeDtypeStruct((n, h), jnp.float32),
    )(p, dacc, w)


def _tc_layer2(p, dacc, w1, wl, n_out, bn=1000):
    d = p.shape[2]
    h = w1.shape[1]
    n = n_out
    c_out = wl.shape[1]

    def body(p_ref, d_ref, w1_ref, wl_ref, o_ref):
        agg = p_ref[0] + p_ref[1]
        deg = d_ref[0, :, 0] + d_ref[1, :, 0]
        inv = 1.0 / jnp.maximum(deg, 1.0)
        aggn = agg * inv[:, None]
        hid = jnp.maximum(
            jnp.dot(aggn, w1_ref[...], preferred_element_type=jnp.float32), 0.0)
        o_ref[...] = jnp.dot(hid, wl_ref[...], preferred_element_type=jnp.float32)

    return pl.pallas_call(
        body,
        grid=(n // bn,),
        in_specs=[
            pl.BlockSpec((NC, bn, d), lambda i: (0, i, 0)),
            pl.BlockSpec((NC, bn, DEG_W), lambda i: (0, i, 0)),
            pl.BlockSpec((d, h), lambda i: (0, 0)),
            pl.BlockSpec((h, c_out), lambda i: (0, 0)),
        ],
        out_specs=pl.BlockSpec((bn, c_out), lambda i: (i, 0)),
        out_shape=jax.ShapeDtypeStruct((n, c_out), jnp.float32),
    )(p, dacc, w1, wl)


def kernel(X, edge_index, W0, W1, W_last):
    n, d = X.shape
    e = edge_index.shape[1]
    n_pad = _pad_rows(n)
    rows_per_tile = n_pad // NS

    z128 = jnp.zeros((rows_per_tile, d), jnp.float32)
    zdeg = jnp.zeros((rows_per_tile, DEG_W), jnp.float32)
    ones = jnp.ones((CHUNK, DEG_W), jnp.float32)

    spmm = _make_spmm(n_pad, e, d)
    degk = _make_deg(n_pad, e, d)

    srcv = edge_index[0]
    dstv = edge_index[1]
    (dacc,) = degk(dstv, zdeg, ones)
    (p1,) = spmm(X, srcv, dstv, z128)
    h1 = _tc_layer1(p1, dacc, W0, n)
    (p2,) = spmm(h1, srcv, dstv, z128)
    out = _tc_layer2(p2, dacc, W1, W_last, n)
    return out


# speedup vs baseline: 1.0088x; 1.0088x over previous
import jax
import jax.numpy as jnp
from jax import lax
from jax.experimental import pallas as pl
from jax.experimental.pallas import tpu as pltpu
from jax.experimental.pallas import tpu_sc as plsc

NC = 2
NS = 16
NW = NC * NS
CHUNK = 80
NBUF = 4
NIDX = 8
F_SLOW = 0.48
DEG_W = 32


def _pad_rows(n):
    return ((n + NS * 8 - 1) // (NS * 8)) * (NS * 8)


def _splits(e):
    chunks = e // CHUNK
    assert chunks * CHUNK == e
    per_pair = chunks // NS
    assert per_pair * NS == chunks
    n1 = max(int(round(per_pair * F_SLOW)), 2)
    n0 = per_pair - n1
    return n0, n1


def _make_spmm(n_pad, e, d):
    rows_per_tile = n_pad // NS
    n0, n1 = _splits(e)

    mesh = plsc.VectorSubcoreMesh(core_axis_name="c", subcore_axis_name="s")
    out_type = [jax.ShapeDtypeStruct((NC, n_pad, d), jnp.float32)]
    scratch = [
        pltpu.VMEM_SHARED((n_pad, d), jnp.float32),
        pltpu.VMEM((NIDX, 2, CHUNK), jnp.int32),
        pltpu.VMEM((NBUF, CHUNK, d), jnp.float32),
        pltpu.SemaphoreType.DMA((NIDX,)),
        pltpu.SemaphoreType.DMA((NBUF,)),
        pltpu.SemaphoreType.DMA((NBUF,)),
    ]

    def body(x_hbm, ei_hbm, z_hbm, out_hbm, acc, idx_v, rows_v, xs, gs, cs):
        c = lax.axis_index("c")
        s = lax.axis_index("s")
        row0 = s * rows_per_tile
        nb = jnp.where(c == 0, n0, n1)
        chunk0 = jnp.where(c == 0, s * n0, NS * n0 + s * n1)
        pltpu.sync_copy(z_hbm, acc.at[pl.ds(row0, rows_per_tile)])
        plsc.subcore_barrier()

        def idx_load(chunk, slot):
            base = (chunk0 + jnp.minimum(chunk, nb - 1)) * CHUNK
            pltpu.make_async_copy(
                ei_hbm.at[:, pl.ds(base, CHUNK)], idx_v.at[slot],
                xs.at[slot]).start()

        def idx_wait(slot):
            pltpu.make_async_copy(
                ei_hbm.at[:, pl.ds(0, CHUNK)], idx_v.at[slot],
                xs.at[slot]).wait()

        def gather_start(slot8, buf):
            pltpu.make_async_copy(
                x_hbm.at[idx_v.at[slot8, 0]], rows_v.at[buf],
                gs.at[buf]).start()

        def gather_wait(buf):
            pltpu.make_async_copy(
                x_hbm.at[idx_v.at[0, 0]], rows_v.at[buf], gs.at[buf]).wait()

        def scatter_start(slot8, buf):
            pltpu.make_async_copy(
                rows_v.at[buf], acc.at[idx_v.at[slot8, 1]],
                cs.at[buf]).start(add=True)

        def scatter_wait(buf):
            pltpu.make_async_copy(
                rows_v.at[buf], acc.at[idx_v.at[0, 1]], cs.at[buf]).wait()

        for k in range(6):
            idx_load(k, k)
        idx_wait(0)
        gather_start(0, 0)
        idx_wait(1)
        gather_start(1, 1)
        for i in (0, 1):
            gather_wait(i)
            scatter_start(i, i)
            idx_load(i + 6, i + 6)
            idx_wait(i + 2)
            gather_start(i + 2, i + 2)

        def step(i, carry):
            b = lax.rem(i, NBUF)
            gather_wait(b)
            scatter_start(lax.rem(i, NIDX), b)
            scatter_wait(lax.rem(i + 2, NBUF))
            idx_load(i + 6, lax.rem(i + 6, NIDX))
            idx_wait(lax.rem(i + 2, NIDX))
            gather_start(lax.rem(i + 2, NIDX), lax.rem(i + 2, NBUF))
            return carry

        lax.fori_loop(2, nb, step, 0)
        scatter_wait(lax.rem(nb - 2, NBUF))
        scatter_wait(lax.rem(nb - 1, NBUF))
        gather_wait(lax.rem(nb, NBUF))
        gather_wait(lax.rem(nb + 1, NBUF))
        for k in range(2, 6):
            idx_wait(lax.rem(nb + k, NIDX))
        plsc.subcore_barrier()
        pltpu.sync_copy(acc.at[pl.ds(row0, rows_per_tile)],
                        out_hbm.at[c, pl.ds(row0, rows_per_tile)])

    return pl.kernel(body, out_type=out_type, mesh=mesh, scratch_types=scratch,
                     compiler_params=pltpu.CompilerParams(use_tc_tiling_on_sc=False))


def _make_deg(n_pad, e, d):
    rows_per_tile = n_pad // NS
    nb = e // (CHUNK * NW)
    assert nb * CHUNK * NW == e and nb >= 2

    mesh = plsc.VectorSubcoreMesh(core_axis_name="c", subcore_axis_name="s")
    out_type = [jax.ShapeDtypeStruct((NC, n_pad, DEG_W), jnp.float32)]
    scratch = [
        pltpu.VMEM_SHARED((n_pad, DEG_W), jnp.float32),
        pltpu.VMEM((NIDX, 2, CHUNK), jnp.int32),
        pltpu.VMEM((CHUNK, DEG_W), jnp.float32),
        pltpu.SemaphoreType.DMA((NIDX,)),
        pltpu.SemaphoreType.DMA((NBUF,)),
    ]

    def body(ei_hbm, z_hbm, ones_hbm, out_hbm, dacc, idx_v, ones_v, xs, cs):
        c = lax.axis_index("c")
        s = lax.axis_index("s")
        wid = s * NC + c
        row0 = s * rows_per_tile
        chunk0 = wid * nb
        pltpu.sync_copy(z_hbm, dacc.at[pl.ds(row0, rows_per_tile)])
        pltpu.sync_copy(ones_hbm, ones_v)
        plsc.subcore_barrier()

        def idx_load(chunk, slot):
            base = (chunk0 + jnp.minimum(chunk, nb - 1)) * CHUNK
            pltpu.make_async_copy(
                ei_hbm.at[:, pl.ds(base, CHUNK)], idx_v.at[slot],
                xs.at[slot]).start()

        def idx_wait(slot):
            pltpu.make_async_copy(
                ei_hbm.at[:, pl.ds(0, CHUNK)], idx_v.at[slot],
                xs.at[slot]).wait()

        def scatter_start(slot8, buf):
            pltpu.make_async_copy(
                ones_v, dacc.at[idx_v.at[slot8, 1]],
                cs.at[buf]).start(add=True)

        def scatter_wait(buf):
            pltpu.make_async_copy(
                ones_v, dacc.at[idx_v.at[0, 1]], cs.at[buf]).wait()

        for k in range(6):
            idx_load(k, k)
        for i in (0, 1):
            idx_wait(i)
            scatter_start(i, i)
            idx_load(i + 6, i + 6)

        def step(i, carry):
            idx_wait(lax.rem(i, NIDX))
            scatter_start(lax.rem(i, NIDX), lax.rem(i, NBUF))
            scatter_wait(lax.rem(i + 2, NBUF))
            idx_load(i + 6, lax.rem(i + 6, NIDX))
            return carry

        lax.fori_loop(2, nb, step, 0)
        scatter_wait(lax.rem(nb - 2, NBUF))
        scatter_wait(lax.rem(nb - 1, NBUF))
        for k in range(0, 6):
            idx_wait(lax.rem(nb + k, NIDX))
        plsc.subcore_barrier()
        pltpu.sync_copy(dacc.at[pl.ds(row0, rows_per_tile)],
                        out_hbm.at[c, pl.ds(row0, rows_per_tile)])

    return pl.kernel(body, out_type=out_type, mesh=mesh, scratch_types=scratch,
                     compiler_params=pltpu.CompilerParams(use_tc_tiling_on_sc=False))


def _tc_layer1(p, dacc, w, n_out, bn=1000):
    d = p.shape[2]
    h = w.shape[1]
    n = n_out

    def body(p_ref, d_ref, w_ref, o_ref):
        agg = p_ref[0] + p_ref[1]
        deg = d_ref[0, :, 0] + d_ref[1, :, 0]
        inv = 1.0 / jnp.maximum(deg, 1.0)
        aggn = agg * inv[:, None]
        o_ref[...] = jnp.maximum(
            jnp.dot(aggn, w_ref[...], preferred_element_type=jnp.float32), 0.0)

    return pl.pallas_call(
        body,
        grid=(n // bn,),
        in_specs=[
            pl.BlockSpec((NC, bn, d), lambda i: (0, i, 0)),
            pl.BlockSpec((NC, bn, DEG_W), lambda i: (0, i, 0)),
            pl.BlockSpec((d, h), lambda i: (0, 0)),
        ],
        out_specs=pl.BlockSpec((bn, h), lambda i: (i, 0)),
        out_shape=jax.ShapeDtypeStruct((n, h), jnp.float32),
    )(p, dacc, w)


def _tc_layer2(p, dacc, w1, wl, n_out, bn=1000):
    d = p.shape[2]
    h = w1.shape[1]
    n = n_out
    c_out = wl.shape[1]

    def body(p_ref, d_ref, w1_ref, wl_ref, o_ref):
        agg = p_ref[0] + p_ref[1]
        deg = d_ref[0, :, 0] + d_ref[1, :, 0]
        inv = 1.0 / jnp.maximum(deg, 1.0)
        aggn = agg * inv[:, None]
        hid = jnp.maximum(
            jnp.dot(aggn, w1_ref[...], preferred_element_type=jnp.float32), 0.0)
        o_ref[...] = jnp.dot(hid, wl_ref[...], preferred_element_type=jnp.float32)

    return pl.pallas_call(
        body,
        grid=(n // bn,),
        in_specs=[
            pl.BlockSpec((NC, bn, d), lambda i: (0, i, 0)),
            pl.BlockSpec((NC, bn, DEG_W), lambda i: (0, i, 0)),
            pl.BlockSpec((d, h), lambda i: (0, 0)),
            pl.BlockSpec((h, c_out), lambda i: (0, 0)),
        ],
        out_specs=pl.BlockSpec((bn, c_out), lambda i: (i, 0)),
        out_shape=jax.ShapeDtypeStruct((n, c_out), jnp.float32),
    )(p, dacc, w1, wl)


def kernel(X, edge_index, W0, W1, W_last):
    n, d = X.shape
    e = edge_index.shape[1]
    n_pad = _pad_rows(n)
    rows_per_tile = n_pad // NS

    z128 = jnp.zeros((rows_per_tile, d), jnp.float32)
    zdeg = jnp.zeros((rows_per_tile, DEG_W), jnp.float32)
    ones = jnp.ones((CHUNK, DEG_W), jnp.float32)

    spmm = _make_spmm(n_pad, e, d)
    degk = _make_deg(n_pad, e, d)

    (dacc,) = degk(edge_index, zdeg, ones)
    (p1,) = spmm(X, edge_index, z128)
    h1 = _tc_layer1(p1, dacc, W0, n)
    (p2,) = spmm(h1, edge_index, z128)
    out = _tc_layer2(p2, dacc, W1, W_last, n)
    return out


# speedup vs baseline: 1.0304x; 1.0214x over previous
import jax
import jax.numpy as jnp
from jax import lax
from jax.experimental import pallas as pl
from jax.experimental.pallas import tpu as pltpu
from jax.experimental.pallas import tpu_sc as plsc

NC = 2
NS = 16
NW = NC * NS
CHUNK = 80
NBUF = 4
NIDX = 8
F_SLOW = 0.50
DEG_W = 32


def _pad_rows(n):
    return ((n + NS * 8 - 1) // (NS * 8)) * (NS * 8)


def _splits(e):
    chunks = e // CHUNK
    assert chunks * CHUNK == e
    per_pair = chunks // NS
    assert per_pair * NS == chunks
    n1 = max(int(round(per_pair * F_SLOW)), 2)
    n0 = per_pair - n1
    return n0, n1


def _make_spmm(n_pad, e, d):
    rows_per_tile = n_pad // NS
    n0, n1 = _splits(e)

    mesh = plsc.VectorSubcoreMesh(core_axis_name="c", subcore_axis_name="s")
    out_type = [jax.ShapeDtypeStruct((NC, n_pad, d), jnp.float32)]
    scratch = [
        pltpu.VMEM_SHARED((n_pad, d), jnp.float32),
        pltpu.VMEM((NIDX, 2, CHUNK), jnp.int32),
        pltpu.VMEM((NBUF, CHUNK, d), jnp.float32),
        pltpu.SemaphoreType.DMA((NIDX,)),
        pltpu.SemaphoreType.DMA((NBUF,)),
        pltpu.SemaphoreType.DMA((NBUF,)),
    ]

    def body(x_hbm, ei_hbm, z_hbm, out_hbm, acc, idx_v, rows_v, xs, gs, cs):
        c = lax.axis_index("c")
        s = lax.axis_index("s")
        row0 = s * rows_per_tile
        nb = jnp.where(c == 0, n0, n1)
        chunk0 = jnp.where(c == 0, s * n0, NS * n0 + s * n1)
        pltpu.sync_copy(z_hbm, acc.at[pl.ds(row0, rows_per_tile)])
        plsc.subcore_barrier()

        def idx_load(chunk, slot):
            base = (chunk0 + jnp.minimum(chunk, nb - 1)) * CHUNK
            pltpu.make_async_copy(
                ei_hbm.at[:, pl.ds(base, CHUNK)], idx_v.at[slot],
                xs.at[slot]).start()

        def idx_wait(slot):
            pltpu.make_async_copy(
                ei_hbm.at[:, pl.ds(0, CHUNK)], idx_v.at[slot],
                xs.at[slot]).wait()

        def gather_start(slot8, buf):
            pltpu.make_async_copy(
                x_hbm.at[idx_v.at[slot8, 0]], rows_v.at[buf],
                gs.at[buf]).start()

        def gather_wait(buf):
            pltpu.make_async_copy(
                x_hbm.at[idx_v.at[0, 0]], rows_v.at[buf], gs.at[buf]).wait()

        def scatter_start(slot8, buf):
            pltpu.make_async_copy(
                rows_v.at[buf], acc.at[idx_v.at[slot8, 1]],
                cs.at[buf]).start(add=True)

        def scatter_wait(buf):
            pltpu.make_async_copy(
                rows_v.at[buf], acc.at[idx_v.at[0, 1]], cs.at[buf]).wait()

        for k in range(6):
            idx_load(k, k)
        idx_wait(0)
        gather_start(0, 0)
        idx_wait(1)
        gather_start(1, 1)
        for i in (0, 1):
            gather_wait(i)
            scatter_start(i, i)
            idx_load(i + 6, i + 6)
            idx_wait(i + 2)
            gather_start(i + 2, i + 2)

        def step(i, carry):
            b = lax.rem(i, NBUF)
            gather_wait(b)
            scatter_start(lax.rem(i, NIDX), b)
            scatter_wait(lax.rem(i + 2, NBUF))
            idx_load(i + 6, lax.rem(i + 6, NIDX))
            idx_wait(lax.rem(i + 2, NIDX))
            gather_start(lax.rem(i + 2, NIDX), lax.rem(i + 2, NBUF))
            return carry

        lax.fori_loop(2, nb, step, 0)
        scatter_wait(lax.rem(nb - 2, NBUF))
        scatter_wait(lax.rem(nb - 1, NBUF))
        gather_wait(lax.rem(nb, NBUF))
        gather_wait(lax.rem(nb + 1, NBUF))
        for k in range(2, 6):
            idx_wait(lax.rem(nb + k, NIDX))
        plsc.subcore_barrier()
        pltpu.sync_copy(acc.at[pl.ds(row0, rows_per_tile)],
                        out_hbm.at[c, pl.ds(row0, rows_per_tile)])

    return pl.kernel(body, out_type=out_type, mesh=mesh, scratch_types=scratch,
                     compiler_params=pltpu.CompilerParams(use_tc_tiling_on_sc=False))


def _make_deg(n_pad, e, d):
    rows_per_tile = n_pad // NS
    nb = e // (CHUNK * NW)
    assert nb * CHUNK * NW == e and nb >= 2

    mesh = plsc.VectorSubcoreMesh(core_axis_name="c", subcore_axis_name="s")
    out_type = [jax.ShapeDtypeStruct((NC, n_pad, DEG_W), jnp.float32)]
    scratch = [
        pltpu.VMEM_SHARED((n_pad, DEG_W), jnp.float32),
        pltpu.VMEM((NIDX, 2, CHUNK), jnp.int32),
        pltpu.VMEM((CHUNK, DEG_W), jnp.float32),
        pltpu.SemaphoreType.DMA((NIDX,)),
        pltpu.SemaphoreType.DMA((NBUF,)),
    ]

    def body(ei_hbm, z_hbm, ones_hbm, out_hbm, dacc, idx_v, ones_v, xs, cs):
        c = lax.axis_index("c")
        s = lax.axis_index("s")
        wid = s * NC + c
        row0 = s * rows_per_tile
        chunk0 = wid * nb
        pltpu.sync_copy(z_hbm, dacc.at[pl.ds(row0, rows_per_tile)])
        pltpu.sync_copy(ones_hbm, ones_v)
        plsc.subcore_barrier()

        def idx_load(chunk, slot):
            base = (chunk0 + jnp.minimum(chunk, nb - 1)) * CHUNK
            pltpu.make_async_copy(
                ei_hbm.at[:, pl.ds(base, CHUNK)], idx_v.at[slot],
                xs.at[slot]).start()

        def idx_wait(slot):
            pltpu.make_async_copy(
                ei_hbm.at[:, pl.ds(0, CHUNK)], idx_v.at[slot],
                xs.at[slot]).wait()

        def scatter_start(slot8, buf):
            pltpu.make_async_copy(
                ones_v, dacc.at[idx_v.at[slot8, 1]],
                cs.at[buf]).start(add=True)

        def scatter_wait(buf):
            pltpu.make_async_copy(
                ones_v, dacc.at[idx_v.at[0, 1]], cs.at[buf]).wait()

        for k in range(6):
            idx_load(k, k)
        for i in (0, 1):
            idx_wait(i)
            scatter_start(i, i)
            idx_load(i + 6, i + 6)

        def step(i, carry):
            idx_wait(lax.rem(i, NIDX))
            scatter_start(lax.rem(i, NIDX), lax.rem(i, NBUF))
            scatter_wait(lax.rem(i + 2, NBUF))
            idx_load(i + 6, lax.rem(i + 6, NIDX))
            return carry

        lax.fori_loop(2, nb, step, 0)
        scatter_wait(lax.rem(nb - 2, NBUF))
        scatter_wait(lax.rem(nb - 1, NBUF))
        for k in range(0, 6):
            idx_wait(lax.rem(nb + k, NIDX))
        plsc.subcore_barrier()
        pltpu.sync_copy(dacc.at[pl.ds(row0, rows_per_tile)],
                        out_hbm.at[c, pl.ds(row0, rows_per_tile)])

    return pl.kernel(body, out_type=out_type, mesh=mesh, scratch_types=scratch,
                     compiler_params=pltpu.CompilerParams(use_tc_tiling_on_sc=False))


def _tc_layer1(p, dacc, w, n_out, bn=1000):
    d = p.shape[2]
    h = w.shape[1]
    n = n_out

    def body(p_ref, d_ref, w_ref, o_ref):
        agg = p_ref[0] + p_ref[1]
        deg = d_ref[0, :, 0] + d_ref[1, :, 0]
        inv = 1.0 / jnp.maximum(deg, 1.0)
        aggn = agg * inv[:, None]
        o_ref[...] = jnp.maximum(
            jnp.dot(aggn, w_ref[...], preferred_element_type=jnp.float32), 0.0)

    return pl.pallas_call(
        body,
        grid=(n // bn,),
        in_specs=[
            pl.BlockSpec((NC, bn, d), lambda i: (0, i, 0)),
            pl.BlockSpec((NC, bn, DEG_W), lambda i: (0, i, 0)),
            pl.BlockSpec((d, h), lambda i: (0, 0)),
        ],
        out_specs=pl.BlockSpec((bn, h), lambda i: (i, 0)),
        out_shape=jax.ShapeDtypeStruct((n, h), jnp.float32),
    )(p, dacc, w)


def _tc_layer2(p, dacc, w1, wl, n_out, bn=1000):
    d = p.shape[2]
    h = w1.shape[1]
    n = n_out
    c_out = wl.shape[1]

    def body(p_ref, d_ref, w1_ref, wl_ref, o_ref):
        agg = p_ref[0] + p_ref[1]
        deg = d_ref[0, :, 0] + d_ref[1, :, 0]
        inv = 1.0 / jnp.maximum(deg, 1.0)
        aggn = agg * inv[:, None]
        hid = jnp.maximum(
            jnp.dot(aggn, w1_ref[...], preferred_element_type=jnp.float32), 0.0)
        o_ref[...] = jnp.dot(hid, wl_ref[...], preferred_element_type=jnp.float32)

    return pl.pallas_call(
        body,
        grid=(n // bn,),
        in_specs=[
            pl.BlockSpec((NC, bn, d), lambda i: (0, i, 0)),
            pl.BlockSpec((NC, bn, DEG_W), lambda i: (0, i, 0)),
            pl.BlockSpec((d, h), lambda i: (0, 0)),
            pl.BlockSpec((h, c_out), lambda i: (0, 0)),
        ],
        out_specs=pl.BlockSpec((bn, c_out), lambda i: (i, 0)),
        out_shape=jax.ShapeDtypeStruct((n, c_out), jnp.float32),
    )(p, dacc, w1, wl)


def kernel(X, edge_index, W0, W1, W_last):
    n, d = X.shape
    e = edge_index.shape[1]
    n_pad = _pad_rows(n)
    rows_per_tile = n_pad // NS

    z128 = jnp.zeros((rows_per_tile, d), jnp.float32)
    zdeg = jnp.zeros((rows_per_tile, DEG_W), jnp.float32)
    ones = jnp.ones((CHUNK, DEG_W), jnp.float32)

    spmm = _make_spmm(n_pad, e, d)
    degk = _make_deg(n_pad, e, d)

    (dacc,) = degk(edge_index, zdeg, ones)
    (p1,) = spmm(X, edge_index, z128)
    h1 = _tc_layer1(p1, dacc, W0, n)
    (p2,) = spmm(h1, edge_index, z128)
    out = _tc_layer2(p2, dacc, W1, W_last, n)
    return out
